# baseline (speedup 1.0000x reference)
import functools

import jax
import jax.numpy as jnp
from jax import lax
from jax.experimental import pallas as pl
from jax.experimental.pallas import tpu as pltpu
from jax.experimental.pallas import tpu_sc as plsc

VOCAB = 1000
VPAD = 1024
EMBED = 128
B = 1024
L = 50

NUM_CORES = 2
NUM_SUBCORES = 16
NW = NUM_CORES * NUM_SUBCORES
NVT = VOCAB // 8
VT_PER_W = 4
NCHUNK = B // 16


def _gram_body(w_ref, wp_ref, b_ref, g_ref):
    w = w_ref[...]
    wp = wp_ref[...]
    g = lax.dot_general(
        w, wp,
        dimension_numbers=(((1,), (1,)), ((), ())),
        preferred_element_type=jnp.float32,
    )
    g_ref[...] = g + b_ref[...]


def _gram(W, Wp, bcol):
    return pl.pallas_call(
        _gram_body,
        out_shape=jax.ShapeDtypeStruct((VOCAB, VPAD), jnp.float32),
    )(W, Wp, bcol)


_sc_mesh = plsc.VectorSubcoreMesh(
    core_axis_name="c", subcore_axis_name="s",
    num_cores=NUM_CORES, num_subcores=NUM_SUBCORES)


def _expand_body(tt_hbm, idst_hbm, out_hbm, ids_v, tt_v, ob0, ob1, sw0, sw1):
    wid = lax.axis_index("s") * NUM_CORES + lax.axis_index("c")
    pltpu.sync_copy(idst_hbm, ids_v)

    def compute(ob, l):
        def cbody(ci, c2):
            for u in range(4):
                off = 64 * ci + 16 * u
                idx16 = ids_v[l, pl.ds(off, 16)]
                for v_ in range(8):
                    x = plsc.load_gather(tt_v, [idx16 + (v_ * VPAD)])
                    ob[v_, pl.ds(off, 16)] = x
            return c2
        lax.fori_loop(0, NCHUNK // 4, cbody, 0)

    def w_pair(ob, sw, l, vt):
        return (ob, out_hbm.at[l, pl.ds(8 * vt, 8)], sw)

    def vt_body(vt):
        pltpu.sync_copy(tt_hbm.at[pl.ds(vt * (8 * VPAD), 8 * VPAD)], tt_v)
        compute(ob0, 0)
        src, dst, sw = w_pair(ob0, sw0, 0, vt)
        pltpu.async_copy(src, dst, sw)
        compute(ob1, 1)
        src, dst, sw = w_pair(ob1, sw1, 1, vt)
        pltpu.async_copy(src, dst, sw)

        def l_pair(jj, c2):
            l0 = 2 * jj
            src, dst, sw = w_pair(ob0, sw0, l0, vt)
            pltpu.make_async_copy(src, dst, sw).wait()
            compute(ob0, l0)
            pltpu.async_copy(src, dst, sw)
            l1 = 2 * jj + 1
            src, dst, sw = w_pair(ob1, sw1, l1, vt)
            pltpu.make_async_copy(src, dst, sw).wait()
            compute(ob1, l1)
            pltpu.async_copy(src, dst, sw)
            return c2
        lax.fori_loop(1, L // 2, l_pair, 0)

        src, dst, sw = w_pair(ob0, sw0, 0, vt)
        pltpu.make_async_copy(src, dst, sw).wait()
        src, dst, sw = w_pair(ob1, sw1, 0, vt)
        pltpu.make_async_copy(src, dst, sw).wait()

    def k_body(k, c2):
        vt = wid + NW * k

        @pl.when(vt < NVT)
        def _():
            vt_body(vt)
        return c2

    lax.fori_loop(0, VT_PER_W, k_body, 0)


_expand = functools.partial(
    pl.kernel,
    mesh=_sc_mesh,
    out_type=jax.ShapeDtypeStruct((L, VOCAB, B), jnp.float32),
    scratch_types=[
        pltpu.VMEM((L, B), jnp.int32),
        pltpu.VMEM((8 * VPAD,), jnp.float32),
        pltpu.VMEM((8, B), jnp.float32),
        pltpu.VMEM((8, B), jnp.float32),
        pltpu.SemaphoreType.DMA,
        pltpu.SemaphoreType.DMA,
    ],
    compiler_params=pltpu.CompilerParams(needs_layout_passes=False),
)(_expand_body)


def kernel(input_ids, W, b):
    ids_t = input_ids.T.astype(jnp.int32)
    Wp = jnp.zeros((VPAD, EMBED), jnp.float32).at[:VOCAB].set(W)
    tt = _gram(W, Wp, b.reshape(VOCAB, 1))
    tt1d = tt.reshape(VOCAB * VPAD)
    out_t = _expand(tt1d, ids_t)
    return jnp.transpose(out_t, (2, 0, 1))


# speedup vs baseline: 3.8580x; 3.8580x over previous
import functools

import jax
import jax.numpy as jnp
from jax import lax
from jax.experimental import pallas as pl
from jax.experimental.pallas import tpu as pltpu
from jax.experimental.pallas import tpu_sc as plsc

VOCAB = 1000
VPAD = 1024
EMBED = 128
B = 1024
L = 50

NUM_CORES = 2
NUM_SUBCORES = 16
NW = NUM_CORES * NUM_SUBCORES
NVT = VOCAB // 8
VT_PER_W = 4
NCHUNK = B // 16


def _gram_body(w_ref, wp_ref, b_ref, g_ref):
    w = w_ref[...]
    wp = wp_ref[...]
    g = lax.dot_general(
        w, wp,
        dimension_numbers=(((1,), (1,)), ((), ())),
        preferred_element_type=jnp.float32,
    )
    g_ref[...] = g + b_ref[...]


def _gram(W, Wp, bcol):
    return pl.pallas_call(
        _gram_body,
        out_shape=jax.ShapeDtypeStruct((VOCAB, VPAD), jnp.float32),
    )(W, Wp, bcol)


_sc_mesh = plsc.VectorSubcoreMesh(
    core_axis_name="c", subcore_axis_name="s",
    num_cores=NUM_CORES, num_subcores=NUM_SUBCORES)


def _expand_body(tt_hbm, idst_hbm, out_hbm, ids_v, tt_v, ob0, ob1, sw0, sw1):
    wid = lax.axis_index("s") * NUM_CORES + lax.axis_index("c")
    pltpu.sync_copy(idst_hbm, ids_v)

    def compute(ob, l):
        @plsc.parallel_loop(0, NCHUNK, step=1, unroll=8)
        def cbody(ci):
            off = 16 * ci
            idx16 = ids_v[l, pl.ds(off, 16)]
            for v_ in range(8):
                x = plsc.load_gather(tt_v, [idx16 + (v_ * VPAD)])
                ob[v_, pl.ds(off, 16)] = x

    def w_pair(ob, sw, l, vt):
        return (ob, out_hbm.at[l, pl.ds(8 * vt, 8)], sw)

    def vt_body(vt):
        pltpu.sync_copy(tt_hbm.at[pl.ds(vt * (8 * VPAD), 8 * VPAD)], tt_v)
        compute(ob0, 0)
        src, dst, sw = w_pair(ob0, sw0, 0, vt)
        pltpu.async_copy(src, dst, sw)
        compute(ob1, 1)
        src, dst, sw = w_pair(ob1, sw1, 1, vt)
        pltpu.async_copy(src, dst, sw)

        def l_pair(jj, c2):
            l0 = 2 * jj
            src, dst, sw = w_pair(ob0, sw0, l0, vt)
            pltpu.make_async_copy(src, dst, sw).wait()
            compute(ob0, l0)
            pltpu.async_copy(src, dst, sw)
            l1 = 2 * jj + 1
            src, dst, sw = w_pair(ob1, sw1, l1, vt)
            pltpu.make_async_copy(src, dst, sw).wait()
            compute(ob1, l1)
            pltpu.async_copy(src, dst, sw)
            return c2
        lax.fori_loop(1, L // 2, l_pair, 0)

        src, dst, sw = w_pair(ob0, sw0, 0, vt)
        pltpu.make_async_copy(src, dst, sw).wait()
        src, dst, sw = w_pair(ob1, sw1, 0, vt)
        pltpu.make_async_copy(src, dst, sw).wait()

    def k_body(k, c2):
        vt = wid + NW * k

        @pl.when(vt < NVT)
        def _():
            vt_body(vt)
        return c2

    lax.fori_loop(0, VT_PER_W, k_body, 0)


_expand = functools.partial(
    pl.kernel,
    mesh=_sc_mesh,
    out_type=jax.ShapeDtypeStruct((L, VOCAB, B), jnp.float32),
    scratch_types=[
        pltpu.VMEM((L, B), jnp.int32),
        pltpu.VMEM((8 * VPAD,), jnp.float32),
        pltpu.VMEM((8, B), jnp.float32),
        pltpu.VMEM((8, B), jnp.float32),
        pltpu.SemaphoreType.DMA,
        pltpu.SemaphoreType.DMA,
    ],
    compiler_params=pltpu.CompilerParams(
        needs_layout_passes=False, disable_bounds_checks=True),
)(_expand_body)


def kernel(input_ids, W, b):
    ids_t = input_ids.T.astype(jnp.int32)
    Wp = jnp.zeros((VPAD, EMBED), jnp.float32).at[:VOCAB].set(W)
    tt = _gram(W, Wp, b.reshape(VOCAB, 1))
    tt1d = tt.reshape(VOCAB * VPAD)
    out_t = _expand(tt1d, ids_t)
    return jnp.transpose(out_t, (2, 0, 1))
